# Initial kernel scaffold; baseline (speedup 1.0000x reference)
import jax
import jax.numpy as jnp
from jax.experimental import pallas as pl


def kernel(node_attr, edge_idx, edge_attr, eW1, eb1, eW2, eb2, nW1, nb1, nW2, nb2):
    raise NotImplementedError("write your pallas kernel here")


# speedup vs baseline: 3.3943x; 3.3943x over previous
import functools

import jax
import jax.numpy as jnp
from jax import lax
from jax.experimental import pallas as pl
from jax.experimental.pallas import tpu as pltpu
from jax.experimental.pallas import tpu_sc as plsc

N = 10000
E = 320000
D = 128
ED = 16
CH = 128
NCHUNK = E // CH
NW = 32

_mesh = lambda: plsc.VectorSubcoreMesh(core_axis_name="c", subcore_axis_name="s")

f32 = jnp.float32
i32 = jnp.int32


def _proj_body(x_ref, w1s_ref, w1d_ref, b1_ref, psrc_ref, pdst_ref):
    x = x_ref[...]
    psrc_ref[...] = jnp.dot(x, w1s_ref[...], preferred_element_type=f32)
    pdst_ref[...] = jnp.dot(x, w1d_ref[...], preferred_element_type=f32) + b1_ref[...]


def _edge_mlp_body(s_ref, ea_ref, w1e_ref, w2_ref, b2_ref, out_ref):
    h = jnp.maximum(s_ref[...] + jnp.dot(ea_ref[...], w1e_ref[...],
                                         preferred_element_type=f32), 0.0)
    out_ref[...] = jnp.dot(h, w2_ref[...], preferred_element_type=f32) + b2_ref[...]


def _node_mlp_body(x_ref, a0_ref, a1_ref, d0_ref, d1_ref,
                   w1a_ref, w1b_ref, b1_ref, w2_ref, b2_ref,
                   upd_ref, em_ref):
    agg = a0_ref[...] + a1_ref[...]
    deg = jnp.maximum(d0_ref[:, :1] + d1_ref[:, :1], 1.0)
    h2 = jnp.maximum(jnp.dot(x_ref[...], w1a_ref[...], preferred_element_type=f32)
                     + jnp.dot(agg, w1b_ref[...], preferred_element_type=f32)
                     + b1_ref[...], 0.0)
    upd_ref[...] = jnp.dot(h2, w2_ref[...], preferred_element_type=f32) + b2_ref[...]
    em = agg / deg
    em_ref[...] = jnp.concatenate([em, jnp.zeros((em.shape[0], D - ED), f32)], axis=1)


def _final_body(upd_ref, m0_ref, m1_ref, d0_ref, d1_ref, out_ref):
    deg = jnp.maximum(d0_ref[:, :1] + d1_ref[:, :1], 1.0)
    out_ref[...] = 0.5 * (upd_ref[...] + (m0_ref[...] + m1_ref[...]) / deg)


def _gather_sum_body(psrc_hbm, pdst_hbm, src_hbm, dst_hbm, s_hbm,
                     sidx, didx, ra, rb, sem_a, sem_b):
    cid = lax.axis_index("c")
    sid = lax.axis_index("s")
    wid = sid * 2 + cid
    n_my = (NCHUNK - wid + NW - 1) // NW

    def body(i, carry):
        base = (wid + i * NW) * CH
        pltpu.sync_copy(src_hbm.at[pl.ds(base, CH)], sidx)
        pltpu.sync_copy(dst_hbm.at[pl.ds(base, CH)], didx.at[0])
        ca = pltpu.async_copy(psrc_hbm.at[sidx], ra, sem_a)
        cb = pltpu.async_copy(pdst_hbm.at[didx.at[0]], rb, sem_b)
        ca.wait()
        cb.wait()

        def add_row(r, c2):
            for j in range(D // 16):
                sl = pl.ds(j * 16, 16)
                ra[r, sl] = ra[r, sl] + rb[r, sl]
            return c2

        lax.fori_loop(0, CH, add_row, 0)
        pltpu.sync_copy(ra, s_hbm.at[pl.ds(base, CH)])
        return carry

    lax.fori_loop(0, n_my, body, 0)


def _sc_gather_sum(psrc, pdst, src, dst):
    k = pl.kernel(
        _gather_sum_body,
        out_type=jax.ShapeDtypeStruct((E, D), f32),
        mesh=_mesh(),
        scratch_types=[
            pltpu.VMEM((CH,), i32),
            pltpu.VMEM((1, CH), i32),
            pltpu.VMEM((CH, D), f32),
            pltpu.VMEM((CH, D), f32),
            pltpu.SemaphoreType.DMA,
            pltpu.SemaphoreType.DMA,
        ],
    )
    return k(psrc, pdst, src, dst)


def _row_split(sid):
    off = sid * 632
    return off


def _scatter_edges_body(upd_hbm, dst_hbm, z16_hbm, agg_out, deg_out,
                        didx, vals, ones_v, agg_sh, deg_sh, sem_v):
    cid = lax.axis_index("c")
    sid = lax.axis_index("s")
    wid = sid * 2 + cid
    n_my = (NCHUNK - wid + NW - 1) // NW

    def fill_ones(r, c2):
        ones_v[r, :] = jnp.ones((16,), f32)
        return c2

    lax.fori_loop(0, CH, fill_ones, 0)

    off = _row_split(sid)

    @pl.when(sid < 15)
    def _():
        pltpu.sync_copy(z16_hbm.at[pl.ds(off, 632)], agg_sh.at[pl.ds(off, 632)])
        pltpu.sync_copy(z16_hbm.at[pl.ds(off, 632)], deg_sh.at[pl.ds(off, 632)])

    @pl.when(sid == 15)
    def _():
        pltpu.sync_copy(z16_hbm.at[pl.ds(9480, 520)], agg_sh.at[pl.ds(9480, 520)])
        pltpu.sync_copy(z16_hbm.at[pl.ds(9480, 520)], deg_sh.at[pl.ds(9480, 520)])

    plsc.subcore_barrier()

    def body(i, carry):
        base = (wid + i * NW) * CH
        pltpu.sync_copy(dst_hbm.at[pl.ds(base, CH)], didx.at[0])
        cv = pltpu.async_copy(upd_hbm.at[pl.ds(base, CH)], vals, sem_v)
        cv.wait()
        pltpu.sync_copy(vals, agg_sh.at[didx.at[0]], add=True)
        pltpu.sync_copy(ones_v, deg_sh.at[didx.at[0]], add=True)
        return carry

    lax.fori_loop(0, n_my, body, 0)
    plsc.subcore_barrier()

    @pl.when(sid < 15)
    def _():
        pltpu.sync_copy(agg_sh.at[pl.ds(off, 632)], agg_out.at[cid, pl.ds(off, 632)])
        pltpu.sync_copy(deg_sh.at[pl.ds(off, 632)], deg_out.at[cid, pl.ds(off, 632)])

    @pl.when(sid == 15)
    def _():
        pltpu.sync_copy(agg_sh.at[pl.ds(9480, 520)], agg_out.at[cid, pl.ds(9480, 520)])
        pltpu.sync_copy(deg_sh.at[pl.ds(9480, 520)], deg_out.at[cid, pl.ds(9480, 520)])


def _sc_scatter_edges(upd_edges, dst, z16):
    k = pl.kernel(
        _scatter_edges_body,
        out_type=(jax.ShapeDtypeStruct((2, N, ED), f32),
                  jax.ShapeDtypeStruct((2, N, ED), f32)),
        mesh=_mesh(),
        scratch_types=[
            pltpu.VMEM((1, CH), i32),
            pltpu.VMEM((CH, ED), f32),
            pltpu.VMEM((CH, ED), f32),
            pltpu.VMEM_SHARED((N, ED), f32),
            pltpu.VMEM_SHARED((N, ED), f32),
            pltpu.SemaphoreType.DMA,
        ],
    )
    return k(upd_edges, dst, z16)


def _finale_body(updn_hbm, em_hbm, ue_hbm, src_hbm, dst_hbm, z128_hbm,
                 nm_out, eo_out,
                 sidx, didx, rows, em_v, ue_v, nm_sh, sem_r, sem_e):
    cid = lax.axis_index("c")
    sid = lax.axis_index("s")
    wid = sid * 2 + cid
    n_my = (NCHUNK - wid + NW - 1) // NW

    off = _row_split(sid)

    @pl.when(sid < 15)
    def _():
        pltpu.sync_copy(z128_hbm.at[pl.ds(off, 632)], nm_sh.at[pl.ds(off, 632)])

    @pl.when(sid == 15)
    def _():
        pltpu.sync_copy(z128_hbm.at[pl.ds(9480, 520)], nm_sh.at[pl.ds(9480, 520)])

    plsc.subcore_barrier()

    def body(i, carry):
        base = (wid + i * NW) * CH
        pltpu.sync_copy(src_hbm.at[pl.ds(base, CH)], sidx)
        pltpu.sync_copy(dst_hbm.at[pl.ds(base, CH)], didx.at[0])
        cr = pltpu.async_copy(updn_hbm.at[sidx], rows, sem_r)
        ce = pltpu.async_copy(em_hbm.at[didx.at[0]], em_v, sem_e)
        cr.wait()
        pltpu.sync_copy(rows, nm_sh.at[didx.at[0]], add=True)
        pltpu.sync_copy(ue_hbm.at[pl.ds(base, CH)], ue_v)
        ce.wait()

        def mix_row(r, c2):
            ue_v[r, :] = 0.5 * (ue_v[r, :] + em_v[r, pl.ds(0, ED)])
            return c2

        lax.fori_loop(0, CH, mix_row, 0)
        pltpu.sync_copy(ue_v, eo_out.at[pl.ds(base, CH)])
        return carry

    lax.fori_loop(0, n_my, body, 0)
    plsc.subcore_barrier()

    @pl.when(sid < 15)
    def _():
        pltpu.sync_copy(nm_sh.at[pl.ds(off, 632)], nm_out.at[cid, pl.ds(off, 632)])

    @pl.when(sid == 15)
    def _():
        pltpu.sync_copy(nm_sh.at[pl.ds(9480, 520)], nm_out.at[cid, pl.ds(9480, 520)])


def _sc_finale(upd_nodes, edge_mean, upd_edges, src, dst, z128):
    k = pl.kernel(
        _finale_body,
        out_type=(jax.ShapeDtypeStruct((2, N, D), f32),
                  jax.ShapeDtypeStruct((E, ED), f32)),
        mesh=_mesh(),
        scratch_types=[
            pltpu.VMEM((CH,), i32),
            pltpu.VMEM((1, CH), i32),
            pltpu.VMEM((CH, D), f32),
            pltpu.VMEM((CH, D), f32),
            pltpu.VMEM((CH, ED), f32),
            pltpu.VMEM_SHARED((N, D), f32),
            pltpu.SemaphoreType.DMA,
            pltpu.SemaphoreType.DMA,
        ],
    )
    return k(upd_nodes, edge_mean, upd_edges, src, dst, z128)


def kernel(node_attr, edge_idx, edge_attr, eW1, eb1, eW2, eb2, nW1, nb1, nW2, nb2):
    src = edge_idx[0]
    dst = edge_idx[1]
    w1s = eW1[:D]
    w1d = eW1[D:2 * D]
    w1e = eW1[2 * D:]
    nw1a = nW1[:D]
    nw1b = nW1[D:]
    eb1r = eb1.reshape(1, D)
    eb2r = eb2.reshape(1, ED)
    nb1r = nb1.reshape(1, D)
    nb2r = nb2.reshape(1, D)
    z16 = jnp.zeros((N, ED), f32)
    z128 = jnp.zeros((N, D), f32)

    TN = 2000
    gn = N // TN
    full = lambda shape: pl.BlockSpec(shape, lambda i: tuple(0 for _ in shape))

    psrc, pdst = pl.pallas_call(
        _proj_body,
        grid=(gn,),
        in_specs=[pl.BlockSpec((TN, D), lambda i: (i, 0)),
                  full((D, D)), full((D, D)), full((1, D))],
        out_specs=[pl.BlockSpec((TN, D), lambda i: (i, 0)),
                   pl.BlockSpec((TN, D), lambda i: (i, 0))],
        out_shape=[jax.ShapeDtypeStruct((N, D), f32),
                   jax.ShapeDtypeStruct((N, D), f32)],
    )(node_attr, w1s, w1d, eb1r)

    s_sum = _sc_gather_sum(psrc, pdst, src, dst)

    TE = 3200
    ge = E // TE
    upd_edges = pl.pallas_call(
        _edge_mlp_body,
        grid=(ge,),
        in_specs=[pl.BlockSpec((TE, D), lambda i: (i, 0)),
                  pl.BlockSpec((TE, ED), lambda i: (i, 0)),
                  full((ED, D)), full((D, ED)), full((1, ED))],
        out_specs=pl.BlockSpec((TE, ED), lambda i: (i, 0)),
        out_shape=jax.ShapeDtypeStruct((E, ED), f32),
    )(s_sum, edge_attr, w1e, eW2, eb2r)

    agg_p, deg_p = _sc_scatter_edges(upd_edges, dst, z16)

    upd_nodes, edge_mean = pl.pallas_call(
        _node_mlp_body,
        grid=(gn,),
        in_specs=[pl.BlockSpec((TN, D), lambda i: (i, 0)),
                  pl.BlockSpec((TN, ED), lambda i: (i, 0)),
                  pl.BlockSpec((TN, ED), lambda i: (i, 0)),
                  pl.BlockSpec((TN, ED), lambda i: (i, 0)),
                  pl.BlockSpec((TN, ED), lambda i: (i, 0)),
                  full((D, D)), full((ED, D)), full((1, D)),
                  full((D, D)), full((1, D))],
        out_specs=[pl.BlockSpec((TN, D), lambda i: (i, 0)),
                   pl.BlockSpec((TN, D), lambda i: (i, 0))],
        out_shape=[jax.ShapeDtypeStruct((N, D), f32),
                   jax.ShapeDtypeStruct((N, D), f32)],
    )(node_attr, agg_p[0], agg_p[1], deg_p[0], deg_p[1],
      nw1a, nw1b, nb1r, nW2, nb2r)

    nm_p, edge_out = _sc_finale(upd_nodes, edge_mean, upd_edges, src, dst, z128)

    node_out = pl.pallas_call(
        _final_body,
        grid=(gn,),
        in_specs=[pl.BlockSpec((TN, D), lambda i: (i, 0)),
                  pl.BlockSpec((TN, D), lambda i: (i, 0)),
                  pl.BlockSpec((TN, D), lambda i: (i, 0)),
                  pl.BlockSpec((TN, ED), lambda i: (i, 0)),
                  pl.BlockSpec((TN, ED), lambda i: (i, 0))],
        out_specs=pl.BlockSpec((TN, D), lambda i: (i, 0)),
        out_shape=jax.ShapeDtypeStruct((N, D), f32),
    )(upd_nodes, nm_p[0], nm_p[1], deg_p[0], deg_p[1])

    return node_out, edge_out


# speedup vs baseline: 3.4647x; 1.0207x over previous
import functools

import jax
import jax.numpy as jnp
from jax import lax
from jax.experimental import pallas as pl
from jax.experimental.pallas import tpu as pltpu
from jax.experimental.pallas import tpu_sc as plsc

N = 10000
E = 320000
D = 128
ED = 16
CH = 128
NCHUNK = E // CH
NW = 32

_mesh = lambda: plsc.VectorSubcoreMesh(core_axis_name="c", subcore_axis_name="s")

f32 = jnp.float32
i32 = jnp.int32


def _proj_body(x_ref, w1s_ref, w1d_ref, b1_ref, psrc_ref, pdst_ref):
    x = x_ref[...]
    psrc_ref[...] = jnp.dot(x, w1s_ref[...], preferred_element_type=f32)
    pdst_ref[...] = jnp.dot(x, w1d_ref[...], preferred_element_type=f32) + b1_ref[...]


def _edge_mlp_body(s_ref, ea_ref, w1e_ref, w2_ref, b2_ref, out_ref):
    h = jnp.maximum(s_ref[...] + jnp.dot(ea_ref[...], w1e_ref[...],
                                         preferred_element_type=f32), 0.0)
    out_ref[...] = jnp.dot(h, w2_ref[...], preferred_element_type=f32) + b2_ref[...]


def _node_mlp_body(x_ref, a0_ref, a1_ref, d0_ref, d1_ref,
                   w1a_ref, w1b_ref, b1_ref, w2_ref, b2_ref,
                   upd_ref, em_ref):
    agg = a0_ref[...] + a1_ref[...]
    deg = jnp.maximum(d0_ref[:, :1] + d1_ref[:, :1], 1.0)
    h2 = jnp.maximum(jnp.dot(x_ref[...], w1a_ref[...], preferred_element_type=f32)
                     + jnp.dot(agg, w1b_ref[...], preferred_element_type=f32)
                     + b1_ref[...], 0.0)
    upd_ref[...] = jnp.dot(h2, w2_ref[...], preferred_element_type=f32) + b2_ref[...]
    em_ref[...] = agg / deg


def _final_body(upd_ref, m0_ref, m1_ref, d0_ref, d1_ref, out_ref):
    deg = jnp.maximum(d0_ref[:, :1] + d1_ref[:, :1], 1.0)
    out_ref[...] = 0.5 * (upd_ref[...] + (m0_ref[...] + m1_ref[...]) / deg)


def _gather_sum_body(psrc_hbm, pdst_hbm, src_hbm, dst_hbm, s_hbm,
                     sidx, didx, ra, rb, sem_a, sem_b):
    cid = lax.axis_index("c")
    sid = lax.axis_index("s")
    wid = sid * 2 + cid
    n_my = (NCHUNK - wid + NW - 1) // NW

    def body(i, carry):
        base = (wid + i * NW) * CH
        pltpu.sync_copy(src_hbm.at[pl.ds(base, CH)], sidx)
        pltpu.sync_copy(dst_hbm.at[pl.ds(base, CH)], didx.at[0])
        ca = pltpu.async_copy(psrc_hbm.at[sidx], ra, sem_a)
        cb = pltpu.async_copy(pdst_hbm.at[didx.at[0]], rb, sem_b)
        ca.wait()
        cb.wait()

        def add_row(r, c2):
            for j in range(D // 16):
                sl = pl.ds(j * 16, 16)
                ra[r, sl] = ra[r, sl] + rb[r, sl]
            return c2

        lax.fori_loop(0, CH, add_row, 0)
        pltpu.sync_copy(ra, s_hbm.at[pl.ds(base, CH)])
        return carry

    lax.fori_loop(0, n_my, body, 0)


def _sc_gather_sum(psrc, pdst, src, dst):
    k = pl.kernel(
        _gather_sum_body,
        out_type=jax.ShapeDtypeStruct((E, D), f32),
        mesh=_mesh(),
        scratch_types=[
            pltpu.VMEM((CH,), i32),
            pltpu.VMEM((1, CH), i32),
            pltpu.VMEM((CH, D), f32),
            pltpu.VMEM((CH, D), f32),
            pltpu.SemaphoreType.DMA,
            pltpu.SemaphoreType.DMA,
        ],
    )
    return k(psrc, pdst, src, dst)


def _row_split(sid):
    off = sid * 632
    return off


def _scatter_edges_body(upd_hbm, dst_hbm, z16_hbm, agg_out, deg_out,
                        didx, vals, ones_v, agg_sh, deg_sh, sem_v):
    cid = lax.axis_index("c")
    sid = lax.axis_index("s")
    wid = sid * 2 + cid
    n_my = (NCHUNK - wid + NW - 1) // NW

    def fill_ones(r, c2):
        ones_v[r, :] = jnp.ones((16,), f32)
        return c2

    lax.fori_loop(0, CH, fill_ones, 0)

    off = _row_split(sid)

    @pl.when(sid < 15)
    def _():
        pltpu.sync_copy(z16_hbm.at[pl.ds(off, 632)], agg_sh.at[pl.ds(off, 632)])
        pltpu.sync_copy(z16_hbm.at[pl.ds(off, 632)], deg_sh.at[pl.ds(off, 632)])

    @pl.when(sid == 15)
    def _():
        pltpu.sync_copy(z16_hbm.at[pl.ds(9480, 520)], agg_sh.at[pl.ds(9480, 520)])
        pltpu.sync_copy(z16_hbm.at[pl.ds(9480, 520)], deg_sh.at[pl.ds(9480, 520)])

    plsc.subcore_barrier()

    def body(i, carry):
        base = (wid + i * NW) * CH
        pltpu.sync_copy(dst_hbm.at[pl.ds(base, CH)], didx.at[0])
        cv = pltpu.async_copy(upd_hbm.at[pl.ds(base, CH)], vals, sem_v)
        cv.wait()
        pltpu.sync_copy(vals, agg_sh.at[didx.at[0]], add=True)
        pltpu.sync_copy(ones_v, deg_sh.at[didx.at[0]], add=True)
        return carry

    lax.fori_loop(0, n_my, body, 0)
    plsc.subcore_barrier()

    @pl.when(sid < 15)
    def _():
        pltpu.sync_copy(agg_sh.at[pl.ds(off, 632)], agg_out.at[cid, pl.ds(off, 632)])
        pltpu.sync_copy(deg_sh.at[pl.ds(off, 632)], deg_out.at[cid, pl.ds(off, 632)])

    @pl.when(sid == 15)
    def _():
        pltpu.sync_copy(agg_sh.at[pl.ds(9480, 520)], agg_out.at[cid, pl.ds(9480, 520)])
        pltpu.sync_copy(deg_sh.at[pl.ds(9480, 520)], deg_out.at[cid, pl.ds(9480, 520)])


def _sc_scatter_edges(upd_edges, dst, z16):
    k = pl.kernel(
        _scatter_edges_body,
        out_type=(jax.ShapeDtypeStruct((2, N, ED), f32),
                  jax.ShapeDtypeStruct((2, N, ED), f32)),
        mesh=_mesh(),
        scratch_types=[
            pltpu.VMEM((1, CH), i32),
            pltpu.VMEM((CH, ED), f32),
            pltpu.VMEM((CH, ED), f32),
            pltpu.VMEM_SHARED((N, ED), f32),
            pltpu.VMEM_SHARED((N, ED), f32),
            pltpu.SemaphoreType.DMA,
        ],
    )
    return k(upd_edges, dst, z16)


def _edge_smooth_body(em_hbm, ue_hbm, dst_hbm, eo_out,
                      didx, em_v, ue_v, em_sh, sem_e):
    cid = lax.axis_index("c")
    sid = lax.axis_index("s")
    wid = sid * 2 + cid
    n_my = (NCHUNK - wid + NW - 1) // NW

    off = _row_split(sid)

    @pl.when(sid < 15)
    def _():
        pltpu.sync_copy(em_hbm.at[pl.ds(off, 632)], em_sh.at[pl.ds(off, 632)])

    @pl.when(sid == 15)
    def _():
        pltpu.sync_copy(em_hbm.at[pl.ds(9480, 520)], em_sh.at[pl.ds(9480, 520)])

    plsc.subcore_barrier()

    def body(i, carry):
        base = (wid + i * NW) * CH
        pltpu.sync_copy(dst_hbm.at[pl.ds(base, CH)], didx.at[0])
        ce = pltpu.async_copy(em_sh.at[didx.at[0]], em_v, sem_e)
        pltpu.sync_copy(ue_hbm.at[pl.ds(base, CH)], ue_v)
        ce.wait()

        def mix_row(r, c2):
            ue_v[r, :] = 0.5 * (ue_v[r, :] + em_v[r, :])
            return c2

        lax.fori_loop(0, CH, mix_row, 0)
        pltpu.sync_copy(ue_v, eo_out.at[pl.ds(base, CH)])
        return carry

    lax.fori_loop(0, n_my, body, 0)


def _sc_edge_smooth(edge_mean, upd_edges, dst):
    k = pl.kernel(
        _edge_smooth_body,
        out_type=jax.ShapeDtypeStruct((E, ED), f32),
        mesh=_mesh(),
        scratch_types=[
            pltpu.VMEM((1, CH), i32),
            pltpu.VMEM((CH, ED), f32),
            pltpu.VMEM((CH, ED), f32),
            pltpu.VMEM_SHARED((N, ED), f32),
            pltpu.SemaphoreType.DMA,
        ],
    )
    return k(edge_mean, upd_edges, dst)


def _node_mean_body(updn_hbm, src_hbm, dst_hbm, z128_hbm, nm_out,
                    sidx, didx, rows, nm_sh, sem_r):
    cid = lax.axis_index("c")
    sid = lax.axis_index("s")
    wid = sid * 2 + cid
    n_my = (NCHUNK - wid + NW - 1) // NW

    off = _row_split(sid)

    @pl.when(sid < 15)
    def _():
        pltpu.sync_copy(z128_hbm.at[pl.ds(off, 632)], nm_sh.at[pl.ds(off, 632)])

    @pl.when(sid == 15)
    def _():
        pltpu.sync_copy(z128_hbm.at[pl.ds(9480, 520)], nm_sh.at[pl.ds(9480, 520)])

    plsc.subcore_barrier()

    def body(i, carry):
        base = (wid + i * NW) * CH
        pltpu.sync_copy(src_hbm.at[pl.ds(base, CH)], sidx)
        pltpu.sync_copy(dst_hbm.at[pl.ds(base, CH)], didx.at[0])
        cr = pltpu.async_copy(updn_hbm.at[sidx], rows, sem_r)
        cr.wait()
        pltpu.sync_copy(rows, nm_sh.at[didx.at[0]], add=True)
        return carry

    lax.fori_loop(0, n_my, body, 0)
    plsc.subcore_barrier()

    @pl.when(sid < 15)
    def _():
        pltpu.sync_copy(nm_sh.at[pl.ds(off, 632)], nm_out.at[cid, pl.ds(off, 632)])

    @pl.when(sid == 15)
    def _():
        pltpu.sync_copy(nm_sh.at[pl.ds(9480, 520)], nm_out.at[cid, pl.ds(9480, 520)])


def _sc_node_mean(upd_nodes, src, dst, z128):
    k = pl.kernel(
        _node_mean_body,
        out_type=jax.ShapeDtypeStruct((2, N, D), f32),
        mesh=_mesh(),
        scratch_types=[
            pltpu.VMEM((CH,), i32),
            pltpu.VMEM((1, CH), i32),
            pltpu.VMEM((CH, D), f32),
            pltpu.VMEM_SHARED((N, D), f32),
            pltpu.SemaphoreType.DMA,
        ],
    )
    return k(upd_nodes, src, dst, z128)


def kernel(node_attr, edge_idx, edge_attr, eW1, eb1, eW2, eb2, nW1, nb1, nW2, nb2):
    src = edge_idx[0]
    dst = edge_idx[1]
    w1s = eW1[:D]
    w1d = eW1[D:2 * D]
    w1e = eW1[2 * D:]
    nw1a = nW1[:D]
    nw1b = nW1[D:]
    eb1r = eb1.reshape(1, D)
    eb2r = eb2.reshape(1, ED)
    nb1r = nb1.reshape(1, D)
    nb2r = nb2.reshape(1, D)
    z16 = jnp.zeros((N, ED), f32)
    z128 = jnp.zeros((N, D), f32)

    TN = 2000
    gn = N // TN
    full = lambda shape: pl.BlockSpec(shape, lambda i: tuple(0 for _ in shape))

    eidx = edge_idx

    psrc, pdst = pl.pallas_call(
        _proj_body,
        grid=(gn,),
        in_specs=[pl.BlockSpec((TN, D), lambda i: (i, 0)),
                  full((D, D)), full((D, D)), full((1, D))],
        out_specs=[pl.BlockSpec((TN, D), lambda i: (i, 0)),
                   pl.BlockSpec((TN, D), lambda i: (i, 0))],
        out_shape=[jax.ShapeDtypeStruct((N, D), f32),
                   jax.ShapeDtypeStruct((N, D), f32)],
    )(node_attr, w1s, w1d, eb1r)

    s_sum = _sc_gather_sum(psrc, pdst, src, dst)

    TE = 3200
    ge = E // TE
    upd_edges = pl.pallas_call(
        _edge_mlp_body,
        grid=(ge,),
        in_specs=[pl.BlockSpec((TE, D), lambda i: (i, 0)),
                  pl.BlockSpec((TE, ED), lambda i: (i, 0)),
                  full((ED, D)), full((D, ED)), full((1, ED))],
        out_specs=pl.BlockSpec((TE, ED), lambda i: (i, 0)),
        out_shape=jax.ShapeDtypeStruct((E, ED), f32),
    )(s_sum, edge_attr, w1e, eW2, eb2r)

    agg_p, deg_p = _sc_scatter_edges(upd_edges, dst, z16)

    upd_nodes, edge_mean = pl.pallas_call(
        _node_mlp_body,
        grid=(gn,),
        in_specs=[pl.BlockSpec((TN, D), lambda i: (i, 0)),
                  pl.BlockSpec((TN, ED), lambda i: (i, 0)),
                  pl.BlockSpec((TN, ED), lambda i: (i, 0)),
                  pl.BlockSpec((TN, ED), lambda i: (i, 0)),
                  pl.BlockSpec((TN, ED), lambda i: (i, 0)),
                  full((D, D)), full((ED, D)), full((1, D)),
                  full((D, D)), full((1, D))],
        out_specs=[pl.BlockSpec((TN, D), lambda i: (i, 0)),
                   pl.BlockSpec((TN, ED), lambda i: (i, 0))],
        out_shape=[jax.ShapeDtypeStruct((N, D), f32),
                   jax.ShapeDtypeStruct((N, ED), f32)],
    )(node_attr, agg_p[0], agg_p[1], deg_p[0], deg_p[1],
      nw1a, nw1b, nb1r, nW2, nb2r)

    edge_out = _sc_edge_smooth(edge_mean, upd_edges, dst)
    nm_p = _sc_node_mean(upd_nodes, src, dst, z128)

    node_out = pl.pallas_call(
        _final_body,
        grid=(gn,),
        in_specs=[pl.BlockSpec((TN, D), lambda i: (i, 0)),
                  pl.BlockSpec((TN, D), lambda i: (i, 0)),
                  pl.BlockSpec((TN, D), lambda i: (i, 0)),
                  pl.BlockSpec((TN, ED), lambda i: (i, 0)),
                  pl.BlockSpec((TN, ED), lambda i: (i, 0))],
        out_specs=pl.BlockSpec((TN, D), lambda i: (i, 0)),
        out_shape=jax.ShapeDtypeStruct((N, D), f32),
    )(upd_nodes, nm_p[0], nm_p[1], deg_p[0], deg_p[1])

    return node_out, edge_out
